# baseline (speedup 1.0000x reference)
import functools

import jax
import jax.numpy as jnp
from jax import lax
from jax.experimental import pallas as pl
from jax.experimental.pallas import tpu as pltpu
from jax.experimental.pallas import tpu_sc as plsc

_INFO = plsc.get_sparse_core_info()
_NC = _INFO.num_cores
_NS = _INFO.num_subcores
_NW = _NC * _NS

_CHUNK = 128


@functools.lru_cache(maxsize=None)
def _make_gather(V, D, B):
    assert B % _NW == 0
    b_per_w = B // _NW
    nchunk = b_per_w // _CHUNK
    mesh = plsc.VectorSubcoreMesh(core_axis_name="c", subcore_axis_name="s")

    @functools.partial(
        pl.kernel,
        mesh=mesh,
        out_type=jax.ShapeDtypeStruct((_NW, D, b_per_w), jnp.float32),
        scratch_types=[
            pltpu.VMEM((b_per_w,), jnp.int32),
            pltpu.VMEM((nchunk, _CHUNK), jnp.int32),
            pltpu.VMEM((D, b_per_w), jnp.float32),
            pltpu.SemaphoreType.DMA,
        ],
        compiler_params=pltpu.CompilerParams(use_tc_tiling_on_sc=False),
    )
    def gather(table_hbm, idx_hbm, out_hbm, idx_v, list_v, out_v, sem):
        w = lax.axis_index("s") * _NC + lax.axis_index("c")
        base = w * b_per_w
        pltpu.sync_copy(idx_hbm.at[pl.ds(base, b_per_w)], idx_v)
        for c in range(nchunk):
            for h in range(_CHUNK // 16):
                sl = pl.ds(c * _CHUNK + h * 16, 16)
                list_v[c, pl.ds(h * 16, 16)] = idx_v[sl]

        def body(f, carry):
            row = table_hbm.at[f]
            copies = [
                pltpu.async_copy(
                    row.at[list_v.at[c]],
                    out_v.at[f, pl.ds(c * _CHUNK, _CHUNK)],
                    sem,
                )
                for c in range(nchunk)
            ]
            for cp in copies:
                cp.wait()
            return carry

        lax.fori_loop(0, D, body, 0)
        pltpu.sync_copy(out_v, out_hbm.at[w])

    return gather


def kernel(input, indices, prestride, poststride, output_elements):
    n = indices.shape[0]
    d = input.shape[-1]
    v = input.shape[0]
    table_t = input.T
    out_w = _make_gather(v, d, n)(table_t, indices)
    out_t = out_w.transpose(1, 0, 2).reshape(d, n)
    return out_t.T.reshape(1, n, d)


# speedup vs baseline: 5.7593x; 5.7593x over previous
import functools

import jax
import jax.numpy as jnp
from jax import lax
from jax.experimental import pallas as pl
from jax.experimental.pallas import tpu as pltpu
from jax.experimental.pallas import tpu_sc as plsc

_INFO = plsc.get_sparse_core_info()
_NC = _INFO.num_cores
_NS = _INFO.num_subcores
_NW = _NC * _NS
_L = 128

_NBUF = 1


@functools.lru_cache(maxsize=None)
def _make_gather(V, D, B):
    assert B % _NW == 0
    b_per_w = B // _NW
    assert b_per_w % _NBUF == 0
    ntail = V % _L
    v_main = V - ntail
    mesh = plsc.VectorSubcoreMesh(core_axis_name="c", subcore_axis_name="s")

    @functools.partial(
        pl.kernel,
        mesh=mesh,
        out_type=jax.ShapeDtypeStruct((D, B), jnp.float32),
        scratch_types=[
            pltpu.SMEM((128,), jnp.int32),
            pltpu.VMEM((b_per_w // 128, 128), jnp.int32),
            pltpu.VMEM_SHARED((_NS, b_per_w // 128, 128), jnp.int32),
            pltpu.VMEM((_NBUF, D, _L), jnp.float32),
            pltpu.VMEM((D, max(ntail, 1)), jnp.float32),
            pltpu.VMEM((D, b_per_w), jnp.float32),
            pltpu.SemaphoreType.DMA((_NBUF,)),
        ],
        compiler_params=pltpu.CompilerParams(needs_layout_passes=False),
    )
    def gather(
        table_hbm,
        idx_hbm,
        out_hbm,
        idx_s,
        idx_v,
        idx_sh,
        blocks_v,
        tail_v,
        out_v,
        sems,
    ):
        sid = lax.axis_index("s")
        w = sid * _NC + lax.axis_index("c")
        base = w * b_per_w
        pltpu.sync_copy(idx_hbm.at[pl.ds(w * (b_per_w // 128), b_per_w // 128)], idx_v)
        pltpu.sync_copy(idx_v, idx_sh.at[sid])
        if ntail:
            pltpu.sync_copy(table_hbm.at[:, pl.ds(v_main, ntail)], tail_v)

        def block_off(j):
            tc = lax.shift_right_logical(idx_s[j], 7)
            return pl.multiple_of(
                jnp.minimum(tc * _L, v_main - _L).astype(jnp.int32), _L
            )

        def fetch(j, b):
            off = block_off(j)
            for tr in range(D // 8):
                pltpu.async_copy(
                    table_hbm.at[pl.ds(tr * 8, 8), pl.ds(off, _L)],
                    blocks_v.at[b, pl.ds(tr * 8, 8)],
                    sems.at[b],
                )

        def drain(b):
            for tr in range(D // 8):
                pltpu.make_async_copy(
                    table_hbm.at[pl.ds(tr * 8, 8), pl.ds(0, _L)],
                    blocks_v.at[b, pl.ds(tr * 8, 8)],
                    sems.at[b],
                ).wait()

        def extract(r, b, j):
            idx = idx_s[r]
            lane_b = jnp.minimum(idx - block_off(r), _L - 1)
            vb = jnp.full((16,), lane_b, jnp.int32)
            if ntail:
                in_tail = jnp.full((16,), idx >= v_main, jnp.bool_)
                lane_t = jnp.clip(idx - v_main, 0, ntail - 1)
                vt = jnp.full((16,), lane_t, jnp.int32)
            col = jnp.full((16,), j, jnp.int32)
            for h in range(D // 16):
                rows = lax.iota(jnp.int32, 16) + (h * 16)
                vals = plsc.load_gather(blocks_v.at[b], [rows, vb])
                if ntail:
                    tvals = plsc.load_gather(tail_v, [rows, vt])
                    vals = jnp.where(in_tail, tvals, vals)
                plsc.store_scatter(out_v, [rows, col], vals)

        def chunk_body(q, carry):
            pltpu.sync_copy(idx_sh.at[sid].at[q], idx_s)

            def body(r, carry2):
                fetch(r, 0)
                drain(0)
                extract(r, 0, q * 128 + r)
                return carry2

            lax.fori_loop(0, 128, body, 0)
            return carry

        lax.fori_loop(0, b_per_w // 128, chunk_body, 0)
        pltpu.sync_copy(out_v, out_hbm.at[:, pl.ds(base, b_per_w)])

    return gather


def kernel(input, indices, prestride, poststride, output_elements):
    n = indices.shape[0]
    d = input.shape[-1]
    table_t = input.T
    idx2 = indices.reshape(n // 128, 128)
    out_t = _make_gather(input.shape[0], d, n)(table_t, idx2)
    return out_t.T.reshape(1, n, d)


# speedup vs baseline: 19.9733x; 3.4680x over previous
import functools

import jax
import jax.numpy as jnp
from jax import lax
from jax.experimental import pallas as pl
from jax.experimental.pallas import tpu as pltpu
from jax.experimental.pallas import tpu_sc as plsc

_INFO = plsc.get_sparse_core_info()
_NC = _INFO.num_cores
_NS = _INFO.num_subcores
_NW = _NC * _NS
_L = 128

_NBUF = 8


@functools.lru_cache(maxsize=None)
def _make_gather(V, D, B):
    assert B % _NW == 0
    b_per_w = B // _NW
    assert b_per_w % _NBUF == 0
    ntail = V % _L
    v_main = V - ntail
    mesh = plsc.VectorSubcoreMesh(core_axis_name="c", subcore_axis_name="s")

    @functools.partial(
        pl.kernel,
        mesh=mesh,
        out_type=jax.ShapeDtypeStruct((D, B), jnp.float32),
        scratch_types=[
            pltpu.SMEM((128,), jnp.int32),
            pltpu.VMEM((b_per_w // 128, 128), jnp.int32),
            pltpu.VMEM_SHARED((_NS, b_per_w // 128, 128), jnp.int32),
            pltpu.VMEM((_NBUF, D, _L), jnp.float32),
            pltpu.VMEM((D, max(ntail, 1)), jnp.float32),
            pltpu.VMEM((D, b_per_w), jnp.float32),
            pltpu.SemaphoreType.DMA((_NBUF,)),
        ],
        compiler_params=pltpu.CompilerParams(needs_layout_passes=False),
    )
    def gather(
        table_hbm,
        idx_hbm,
        out_hbm,
        idx_s,
        idx_v,
        idx_sh,
        blocks_v,
        tail_v,
        out_v,
        sems,
    ):
        sid = lax.axis_index("s")
        w = sid * _NC + lax.axis_index("c")
        base = w * b_per_w
        pltpu.sync_copy(idx_hbm.at[pl.ds(w * (b_per_w // 128), b_per_w // 128)], idx_v)
        pltpu.sync_copy(idx_v, idx_sh.at[sid])
        if ntail:
            pltpu.sync_copy(table_hbm.at[:, pl.ds(v_main, ntail)], tail_v)

        def block_off(j):
            tc = lax.shift_right_logical(idx_s[j], 7)
            return pl.multiple_of(
                jnp.minimum(tc * _L, v_main - _L).astype(jnp.int32), _L
            )

        def fetch(j, b):
            off = block_off(j)
            for tr in range(D // 8):
                pltpu.async_copy(
                    table_hbm.at[pl.ds(tr * 8, 8), pl.ds(off, _L)],
                    blocks_v.at[b, pl.ds(tr * 8, 8)],
                    sems.at[b],
                )

        def drain(b):
            for tr in range(D // 8):
                pltpu.make_async_copy(
                    table_hbm.at[pl.ds(tr * 8, 8), pl.ds(0, _L)],
                    blocks_v.at[b, pl.ds(tr * 8, 8)],
                    sems.at[b],
                ).wait()

        def extract(r, b, j):
            idx = idx_s[r]
            lane_b = jnp.minimum(idx - block_off(r), _L - 1)
            vb = jnp.full((16,), lane_b, jnp.int32)
            if ntail:
                in_tail = jnp.full((16,), idx >= v_main, jnp.bool_)
                lane_t = jnp.clip(idx - v_main, 0, ntail - 1)
                vt = jnp.full((16,), lane_t, jnp.int32)
            col = jnp.full((16,), j, jnp.int32)
            for h in range(D // 16):
                rows = lax.iota(jnp.int32, 16) + (h * 16)
                vals = plsc.load_gather(blocks_v.at[b], [rows, vb])
                if ntail:
                    tvals = plsc.load_gather(tail_v, [rows, vt])
                    vals = jnp.where(in_tail, tvals, vals)
                plsc.store_scatter(out_v, [rows, col], vals)

        def chunk_body(q, carry):
            pltpu.sync_copy(idx_sh.at[sid].at[q], idx_s)
            for b in range(_NBUF):
                fetch(b, b)

            def body(g, carry2):
                r = g * _NBUF
                for b in range(_NBUF):
                    drain(b)
                    extract(r + b, b, q * 128 + r + b)
                    nxt = r + b + _NBUF

                    @pl.when(nxt < 128)
                    def _():
                        fetch(nxt, b)

                return carry2

            lax.fori_loop(0, 128 // _NBUF, body, 0)
            return carry

        lax.fori_loop(0, b_per_w // 128, chunk_body, 0)
        pltpu.sync_copy(out_v, out_hbm.at[:, pl.ds(base, b_per_w)])

    return gather


def kernel(input, indices, prestride, poststride, output_elements):
    n = indices.shape[0]
    d = input.shape[-1]
    table_t = input.T
    idx2 = indices.reshape(n // 128, 128)
    out_t = _make_gather(input.shape[0], d, n)(table_t, idx2)
    return out_t.T.reshape(1, n, d)
